# Initial kernel scaffold; baseline (speedup 1.0000x reference)
import jax
import jax.numpy as jnp
from jax.experimental import pallas as pl


def kernel(x, edge_index, adj_vals, b, d, f, h, a, c, e, g, alpha, beta, gamma, delta):
    raise NotImplementedError("write your pallas kernel here")


# speedup vs baseline: 4.3325x; 4.3325x over previous
import functools

import jax
import jax.numpy as jnp
from jax import lax
from jax.experimental import pallas as pl
from jax.experimental.pallas import tpu as pltpu
from jax.experimental.pallas import tpu_sc as plsc

N = 10000
E = 320000
D = 128
UNITS = 4
EPS = 1e-06

NC = 2
NS = 16
L = 16
CHUNK = 128
NCHUNK = -(-E // (NS * CHUNK))
EPT = NCHUNK * CHUNK
E_PAD = EPT * NS
ZROWS = 80
NWC = N // ZROWS
TPASS = -(-NWC // NS)


def _pre_body(x_ref, eb, ed, ef, eh, tb, td, tf, th):
    s = jnp.clip(jax.nn.sigmoid(x_ref[...]), EPS, 1.0 - EPS)
    ls = jnp.log(s)
    ln = jnp.log(1.0 - s)
    tb[...] = jnp.exp(eb[...] * ls)
    td[...] = jnp.exp(ed[...] * ln)
    tf[...] = jnp.exp(ef[...] * ls)
    th[...] = jnp.exp(eh[...] * ln)


def _pre(x, eb, ed, ef, eh):
    blk = 1000
    bs_x = pl.BlockSpec((blk, D), lambda i: (i, 0))
    bs_p = pl.BlockSpec((1, D), lambda i: (0, 0))
    return pl.pallas_call(
        _pre_body,
        grid=(N // blk,),
        in_specs=[bs_x, bs_p, bs_p, bs_p, bs_p],
        out_specs=[bs_x] * 4,
        out_shape=[jax.ShapeDtypeStruct((N, D), jnp.float32)] * 4,
    )(x, eb, ed, ef, eh)


def _post_body(sb, sd, sf, sh, ea, ec, ee, eg, al, be, ga, de, out):
    x_ab = jnp.exp(ea[...] * jnp.log(sb[...] + EPS)) * al[...]
    x_cd = jnp.exp(ec[...] * jnp.log(sd[...] + EPS)) * be[...]
    x_ef = jnp.exp(ee[...] * jnp.log(sf[...] + EPS)) * ga[...]
    x_gh = jnp.exp(eg[...] * jnp.log(sh[...] + EPS)) * de[...]
    den = x_ef + x_gh
    out[...] = (x_ab + x_cd) * den / (den * den + 0.001)


def _post(sb, sd, sf, sh, ea, ec, ee, eg, al, be, ga, de):
    blk = 1000
    bs_x = pl.BlockSpec((blk, D), lambda i: (i, 0))
    bs_p = pl.BlockSpec((1, D), lambda i: (0, 0))
    return pl.pallas_call(
        _post_body,
        grid=(N // blk,),
        in_specs=[bs_x] * 4 + [bs_p] * 8,
        out_specs=bs_x,
        out_shape=jax.ShapeDtypeStruct((N, D), jnp.float32),
    )(sb, sd, sf, sh, ea, ec, ee, eg, al, be, ga, de)


def _spmm4(tb, td, tf, th, src, dst, vals):
    mesh = plsc.VectorSubcoreMesh(core_axis_name="c", subcore_axis_name="s")
    out_type = [jax.ShapeDtypeStruct((N, D), jnp.float32) for _ in range(4)]
    scratch = [
        pltpu.VMEM_SHARED((N, D), jnp.float32),
        pltpu.VMEM((CHUNK,), jnp.int32),
        pltpu.VMEM((1, CHUNK), jnp.int32),
        pltpu.VMEM((1, CHUNK), jnp.float32),
        pltpu.VMEM((CHUNK, D), jnp.float32),
        pltpu.VMEM((ZROWS, D), jnp.float32),
        pltpu.VMEM((ZROWS, D), jnp.float32),
        pltpu.SemaphoreType.DMA,
    ]

    @functools.partial(pl.kernel, out_type=out_type, mesh=mesh,
                       scratch_types=scratch)
    def k(tb_h, td_h, tf_h, th_h, src_h, dst_h, val_h,
          ob_h, od_h, of_h, oh_h,
          acc, sidx, didx, vbuf, rows, zbuf, bbuf, sem):
        cid = lax.axis_index("c")
        sid = lax.axis_index("s")

        def zrow(r, _):
            for j in range(D // L):
                zbuf[r, pl.ds(j * L, L)] = jnp.zeros((L,), jnp.float32)
            return 0
        lax.fori_loop(0, ZROWS, zrow, 0)

        tabs = [tb_h, td_h, tf_h, th_h]
        outs = [ob_h, od_h, of_h, oh_h]

        for v in range(4):
            def run(v=v):
                for t in range(TPASS):
                    cidx = sid + NS * t

                    def zinit(cidx=cidx):
                        pltpu.sync_copy(zbuf, acc.at[pl.ds(cidx * ZROWS, ZROWS)])
                    pl.when(cidx < NWC)(zinit)
                plsc.subcore_barrier()

                def chunk(i, _):
                    base = sid * EPT + i * CHUNK
                    pltpu.sync_copy(src_h.at[pl.ds(base, CHUNK)], sidx)
                    pltpu.sync_copy(dst_h.at[pl.ds(base, CHUNK)], didx.at[0])
                    pltpu.sync_copy(val_h.at[pl.ds(base, CHUNK)], vbuf.at[0])
                    pltpu.async_copy(tabs[v].at[sidx], rows, sem).wait()

                    def mul(g, _2):
                        vv = vbuf[0, pl.ds(g * L, L)]
                        for i in range(L):
                            sval = vv[i]
                            for j in range(D // L):
                                rows[g * L + i, pl.ds(j * L, L)] = (
                                    rows[g * L + i, pl.ds(j * L, L)] * sval)
                        return 0
                    lax.fori_loop(0, CHUNK // L, mul, 0)

                    pltpu.sync_copy(rows, acc.at[didx.at[0]], add=True)
                    return 0
                lax.fori_loop(0, NCHUNK, chunk, 0)
                plsc.subcore_barrier()

                for t in range(TPASS):
                    cidx = sid + NS * t

                    def wback(cidx=cidx, v=v):
                        r0 = cidx * ZROWS
                        pltpu.sync_copy(acc.at[pl.ds(r0, ZROWS)], bbuf)
                        pltpu.sync_copy(bbuf, outs[v].at[pl.ds(r0, ZROWS)])
                    pl.when(cidx < NWC)(wback)
                plsc.subcore_barrier()
            pl.when(cid == v // 2)(run)

    return k(tb, td, tf, th, src, dst, vals)


def _rep(p, relu=True):
    q = p.reshape(UNITS)
    if relu:
        q = jax.nn.relu(q)
    return jnp.repeat(q, D // UNITS).reshape(1, D)


def kernel(x, edge_index, adj_vals, b, d, f, h, a, c, e, g,
           alpha, beta, gamma, delta):
    eb, ed, ef, eh = _rep(b), _rep(d), _rep(f), _rep(h)
    ea, ec, ee, eg = _rep(a), _rep(c), _rep(e), _rep(g)
    al, be, ga, de = (_rep(alpha, False), _rep(beta, False),
                      _rep(gamma, False), _rep(delta, False))

    tb, td, tf, th = _pre(x, eb, ed, ef, eh)

    pad = E_PAD - E
    src = jnp.concatenate(
        [edge_index[0].astype(jnp.int32), jnp.zeros((pad,), jnp.int32)])
    dst = jnp.concatenate(
        [edge_index[1].astype(jnp.int32), jnp.zeros((pad,), jnp.int32)])
    vals = jnp.concatenate([adj_vals, jnp.zeros((pad,), jnp.float32)])

    sb, sd, sf, sh = _spmm4(tb, td, tf, th, src, dst, vals)
    return _post(sb, sd, sf, sh, ea, ec, ee, eg, al, be, ga, de)


# speedup vs baseline: 4.8280x; 1.1144x over previous
import functools

import jax
import jax.numpy as jnp
from jax import lax
from jax.experimental import pallas as pl
from jax.experimental.pallas import tpu as pltpu
from jax.experimental.pallas import tpu_sc as plsc

N = 10000
E = 320000
D = 128
UNITS = 4
EPS = 1e-06

NC = 2
NS = 16
L = 16
CHUNK = 128
SUP = 16
NPAIR = SUP // 2
NCHUNK = -(-E // (NS * CHUNK * SUP)) * SUP
EPT = NCHUNK * CHUNK
E_PAD = EPT * NS
NSUP = NCHUNK // SUP
ROWS_T = NS * NCHUNK
ZROWS = 40
NWC = N // ZROWS
TPASS = -(-NWC // NS)
NBLK = 10


def _pre_body(x_ref, eb, ed, ef, eh, t0, t1):
    first = pl.program_id(0) < NBLK
    s = jnp.clip(jax.nn.sigmoid(x_ref[...]), EPS, 1.0 - EPS)
    lsel = jnp.where(first, jnp.log(s), jnp.log(1.0 - s))
    e0 = jnp.where(first, eb[...], ed[...])
    e1 = jnp.where(first, ef[...], eh[...])
    t0[...] = jnp.exp(e0 * lsel)
    t1[...] = jnp.exp(e1 * lsel)


def _pre(x, eb, ed, ef, eh):
    blk = N // NBLK
    bs_x = pl.BlockSpec((blk, D), lambda i: (i % NBLK, 0))
    bs_p = pl.BlockSpec((1, D), lambda i: (0, 0))
    bs_o = pl.BlockSpec((blk, D), lambda i: (i, 0))
    return pl.pallas_call(
        _pre_body,
        grid=(2 * NBLK,),
        in_specs=[bs_x, bs_p, bs_p, bs_p, bs_p],
        out_specs=[bs_o, bs_o],
        out_shape=[jax.ShapeDtypeStruct((2 * N, D), jnp.float32)] * 2,
    )(x, eb, ed, ef, eh)


def _post_body(sb, sd, sf, sh, ea, ec, ee, eg, al, be, ga, de, out):
    x_ab = jnp.exp(ea[...] * jnp.log(sb[...] + EPS)) * al[...]
    x_cd = jnp.exp(ec[...] * jnp.log(sd[...] + EPS)) * be[...]
    x_ef = jnp.exp(ee[...] * jnp.log(sf[...] + EPS)) * ga[...]
    x_gh = jnp.exp(eg[...] * jnp.log(sh[...] + EPS)) * de[...]
    den = x_ef + x_gh
    out[...] = (x_ab + x_cd) * den / (den * den + 0.001)


def _post(o0, o1, ea, ec, ee, eg, al, be, ga, de):
    blk = N // NBLK
    bs_lo = pl.BlockSpec((blk, D), lambda i: (i, 0))
    bs_hi = pl.BlockSpec((blk, D), lambda i: (i + NBLK, 0))
    bs_p = pl.BlockSpec((1, D), lambda i: (0, 0))
    return pl.pallas_call(
        _post_body,
        grid=(NBLK,),
        in_specs=[bs_lo, bs_hi, bs_lo, bs_hi] + [bs_p] * 8,
        out_specs=bs_lo,
        out_shape=jax.ShapeDtypeStruct((N, D), jnp.float32),
    )(o0, o0, o1, o1, ea, ec, ee, eg, al, be, ga, de)


def _spmm4(t0, t1, src2, dst2, val2):
    mesh = plsc.VectorSubcoreMesh(core_axis_name="c", subcore_axis_name="s")
    out_type = [jax.ShapeDtypeStruct((2 * N, D), jnp.float32)] * 2
    scratch = [
        pltpu.VMEM_SHARED((N, D), jnp.float32),
        pltpu.VMEM((SUP, CHUNK), jnp.int32),
        pltpu.VMEM((SUP, CHUNK), jnp.int32),
        pltpu.VMEM((SUP, CHUNK), jnp.float32),
        pltpu.VMEM((SUP, CHUNK), jnp.int32),
        pltpu.VMEM((SUP, CHUNK), jnp.int32),
        pltpu.VMEM((SUP, CHUNK), jnp.float32),
        pltpu.VMEM((CHUNK, D), jnp.float32),
        pltpu.VMEM((CHUNK, D), jnp.float32),
        pltpu.VMEM((ZROWS, D), jnp.float32),
        pltpu.SemaphoreType.DMA,
        pltpu.SemaphoreType.DMA,
        pltpu.SemaphoreType.DMA,
        pltpu.SemaphoreType.DMA,
    ]

    @functools.partial(pl.kernel, out_type=out_type, mesh=mesh,
                       scratch_types=scratch)
    def k(t0_h, t1_h, src_h, dst_h, val_h, o0_h, o1_h,
          acc, sx0, dx0, vl0, sx1, dx1, vl1, rowsA, rowsB, bbuf,
          gsA, gsB, isem0, isem1):
        cid = lax.axis_index("c")
        sid = lax.axis_index("s")
        sets = ((sx0, dx0, vl0, isem0), (sx1, dx1, vl1, isem1))

        def zrow(r, _):
            for j in range(D // L):
                bbuf[r, pl.ds(j * L, L)] = jnp.zeros((L,), jnp.float32)
            return 0

        def scale(buf, vl, j):
            def mul(g, _2):
                vv = vl[j, pl.ds(g * L, L)]
                for i in range(L):
                    sval = vv[i]
                    for qq in range(D // L):
                        buf[g * L + i, pl.ds(qq * L, L)] = (
                            buf[g * L + i, pl.ds(qq * L, L)] * sval)
                return 0
            lax.fori_loop(0, CHUNK // L, mul, 0)

        def core_run(tab, out):
            def idx_issue(s_iter, q, st):
                sx, dx, vl, isem = st
                crow = sid * NCHUNK + s_iter * SUP
                pltpu.async_copy(
                    src_h.at[pl.ds(q * ROWS_T + crow, SUP)], sx, isem)
                pltpu.async_copy(dst_h.at[pl.ds(crow, SUP)], dx, isem)
                pltpu.async_copy(val_h.at[pl.ds(crow, SUP)], vl, isem)

            def idx_wait(st):
                sx, dx, vl, isem = st
                pltpu.make_async_copy(src_h.at[pl.ds(0, SUP)], sx, isem).wait()
                pltpu.make_async_copy(dst_h.at[pl.ds(0, SUP)], dx, isem).wait()
                pltpu.make_async_copy(val_h.at[pl.ds(0, SUP)], vl, isem).wait()

            def gst(idx_ref, buf, sem):
                return pltpu.async_copy(tab.at[idx_ref], buf, sem)

            def qpass(q, _):
                lax.fori_loop(0, ZROWS, zrow, 0)
                for t in range(TPASS):
                    cidx = sid + NS * t

                    def zinit(cidx=cidx):
                        pltpu.sync_copy(
                            bbuf, acc.at[pl.ds(cidx * ZROWS, ZROWS)])
                    pl.when(cidx < NWC)(zinit)
                plsc.subcore_barrier()

                idx_issue(0, q, sets[0])
                idx_wait(sets[0])
                gst(sets[0][0].at[0], rowsA, gsA)
                idx_issue(1, q, sets[1])

                def super_body(s_iter, _):
                    def with_set(cur, nxt):
                        sx, dx, vl, _isem = cur

                        def pair(jp, _2):
                            j0 = 2 * jp
                            j1 = 2 * jp + 1
                            gst(sx.at[j1], rowsB, gsB)
                            pltpu.make_async_copy(
                                tab.at[sx.at[0]], rowsA, gsA).wait()
                            scale(rowsA, vl, j0)
                            pltpu.sync_copy(
                                rowsA, acc.at[dx.at[j0]], add=True)

                            def prefA():
                                gst(sx.at[j0 + 2], rowsA, gsA)
                            pl.when(jp + 1 < NPAIR)(prefA)
                            pltpu.make_async_copy(
                                tab.at[sx.at[0]], rowsB, gsB).wait()
                            scale(rowsB, vl, j1)
                            pltpu.sync_copy(
                                rowsB, acc.at[dx.at[j1]], add=True)
                            return 0
                        lax.fori_loop(0, NPAIR, pair, 0)

                        def boundary():
                            idx_wait(nxt)
                            gst(nxt[0].at[0], rowsA, gsA)

                            def refill():
                                idx_issue(s_iter + 2, q, cur)
                            pl.when(s_iter + 2 < NSUP)(refill)
                        pl.when(s_iter + 1 < NSUP)(boundary)

                    def even():
                        with_set(sets[0], sets[1])

                    def odd():
                        with_set(sets[1], sets[0])
                    pl.when(s_iter % 2 == 0)(even)
                    pl.when(s_iter % 2 == 1)(odd)
                    return 0
                lax.fori_loop(0, NSUP, super_body, 0)

                plsc.subcore_barrier()
                for t in range(TPASS):
                    cidx = sid + NS * t

                    def wback(cidx=cidx):
                        r0 = q * N + cidx * ZROWS
                        pltpu.sync_copy(acc.at[pl.ds(cidx * ZROWS, ZROWS)],
                                        bbuf)
                        pltpu.sync_copy(bbuf, out.at[pl.ds(r0, ZROWS)])
                    pl.when(cidx < NWC)(wback)
                plsc.subcore_barrier()
                return 0
            lax.fori_loop(0, 2, qpass, 0)

        def core0():
            core_run(t0_h, o0_h)
        def core1():
            core_run(t1_h, o1_h)
        pl.when(cid == 0)(core0)
        pl.when(cid == 1)(core1)

    return k(t0, t1, src2, dst2, val2)


def _rep(p, relu=True):
    q = p.reshape(UNITS)
    if relu:
        q = jax.nn.relu(q)
    return jnp.repeat(q, D // UNITS).reshape(1, D)


def kernel(x, edge_index, adj_vals, b, d, f, h, a, c, e, g,
           alpha, beta, gamma, delta):
    eb, ed, ef, eh = _rep(b), _rep(d), _rep(f), _rep(h)
    ea, ec, ee, eg = _rep(a), _rep(c), _rep(e), _rep(g)
    al, be, ga, de = (_rep(alpha, False), _rep(beta, False),
                      _rep(gamma, False), _rep(delta, False))

    t0, t1 = _pre(x, eb, ed, ef, eh)

    pad = E_PAD - E
    src = jnp.concatenate(
        [edge_index[0].astype(jnp.int32), jnp.zeros((pad,), jnp.int32)]
    ).reshape(ROWS_T, CHUNK)
    src2 = jnp.concatenate([src, src + N])
    dst2 = jnp.concatenate(
        [edge_index[1].astype(jnp.int32), jnp.zeros((pad,), jnp.int32)]
    ).reshape(ROWS_T, CHUNK)
    val2 = jnp.concatenate(
        [adj_vals, jnp.zeros((pad,), jnp.float32)]
    ).reshape(ROWS_T, CHUNK)

    o0, o1 = _spmm4(t0, t1, src2, dst2, val2)
    return _post(o0, o1, ea, ec, ee, eg, al, be, ga, de)
